# baseline (speedup 1.0000x reference)
import functools

import jax
import jax.numpy as jnp
from jax import lax
from jax.experimental import pallas as pl
from jax.experimental.pallas import tpu as pltpu
from jax.experimental.pallas import tpu_sc as plsc

NUM_CLASSES = 1000
ROWS = 4096
COLS = 20
L = 16
NW = 32
SLAB = ROWS // NW
TRPC = 25
KPC = 8 * TRPC
CPJ = NUM_CLASSES // KPC
NCHUNK = COLS * CPJ
GRP = SLAB // L


def _make_sc_one_hot():
    mesh = plsc.VectorSubcoreMesh(core_axis_name="c", subcore_axis_name="s")

    @functools.partial(
        pl.kernel,
        mesh=mesh,
        compiler_params=pltpu.CompilerParams(needs_layout_passes=False),
        out_type=jax.ShapeDtypeStruct((COLS, NUM_CLASSES, ROWS), jnp.float32),
        scratch_types=[
            pltpu.VMEM((COLS, SLAB), jnp.int32),
            pltpu.VMEM((KPC, SLAB), jnp.float32),
            pltpu.VMEM((KPC, SLAB), jnp.float32),
            pltpu.VMEM((KPC, SLAB), jnp.float32),
            pltpu.VMEM((KPC, SLAB), jnp.float32),
            pltpu.SemaphoreType.DMA,
            pltpu.SemaphoreType.DMA,
            pltpu.SemaphoreType.DMA,
            pltpu.SemaphoreType.DMA,
        ],
    )
    def k(xt_hbm, out_hbm, idb, buf0, buf1, buf2, buf3, sem0, sem1, sem2, sem3):
        wid = lax.axis_index("s") * 2 + lax.axis_index("c")
        d0_base = wid * SLAB

        pltpu.sync_copy(xt_hbm.at[:, pl.ds(d0_base, SLAB)], idb)

        iota = lax.iota(jnp.int32, L)

        z = jnp.zeros((L,), jnp.float32)

        def zbody(i, _):
            for buf in (buf0, buf1, buf2, buf3):
                for s in range(SLAB // L):
                    buf[i, pl.ds(s * L, L)] = z
            return 0

        lax.fori_loop(0, KPC, zbody, 0)

        def scatter(buf, q, value):
            j = q // CPJ
            kbase = (q % CPJ) * KPC
            vals = jnp.full((L,), value, jnp.float32)
            for g in range(GRP):
                ids = idb[j, pl.ds(g * L, L)]
                r = ids - kbase
                match = (r >= 0) & (r < KPC)
                plsc.store_scatter(
                    buf, [jnp.where(match, r, 0), iota + g * L], vals, mask=match
                )

        NB = 4
        bufs = (buf0, buf1, buf2, buf3)
        sems = (sem0, sem1, sem2, sem3)

        def dst(q):
            j = q // CPJ
            kbase = (q % CPJ) * KPC
            return out_hbm.at[j, pl.ds(kbase, KPC), pl.ds(d0_base, SLAB)]

        def fire(q, buf, sem):
            pltpu.async_copy(buf, dst(q), sem)

        def drain(q, buf, sem):
            pltpu.make_async_copy(buf, dst(q), sem).wait()

        for b in range(NB):
            scatter(bufs[b], jnp.int32(b), 1.0)
            fire(jnp.int32(b), bufs[b], sems[b])

        def body(g, _):
            for b in range(NB):
                q = g + b
                drain(q - NB, bufs[b], sems[b])
                scatter(bufs[b], q - NB, 0.0)
                scatter(bufs[b], q, 1.0)
                fire(q, bufs[b], sems[b])
            return 0

        lax.fori_loop(1, NCHUNK // NB, lambda g, s: body(g * NB, s), 0)

        for b in range(NB):
            drain(jnp.int32(NCHUNK - NB + b), bufs[b], sems[b])

    return k


_sc_one_hot = _make_sc_one_hot()


def kernel(x):
    xt = jnp.transpose(x.astype(jnp.int32))
    out_t = _sc_one_hot(xt)
    return jnp.transpose(out_t, (2, 0, 1))


# speedup vs baseline: 1.0409x; 1.0409x over previous
import functools

import jax
import jax.numpy as jnp
from jax import lax
from jax.experimental import pallas as pl
from jax.experimental.pallas import tpu as pltpu
from jax.experimental.pallas import tpu_sc as plsc

NUM_CLASSES = 1000
ROWS = 4096
COLS = 20
L = 16
NW = 32
SLAB = ROWS // NW
TRPC = 25
KPC = 8 * TRPC
CPJ = NUM_CLASSES // KPC
NCHUNK = COLS * CPJ
GRP = SLAB // L


def _make_sc_one_hot():
    mesh = plsc.VectorSubcoreMesh(core_axis_name="c", subcore_axis_name="s")

    @functools.partial(
        pl.kernel,
        mesh=mesh,
        compiler_params=pltpu.CompilerParams(needs_layout_passes=False),
        out_type=jax.ShapeDtypeStruct((COLS, NUM_CLASSES, ROWS), jnp.float32),
        scratch_types=[
            pltpu.VMEM((COLS, SLAB), jnp.int32),
            pltpu.VMEM((KPC, SLAB), jnp.float32),
            pltpu.VMEM((KPC, SLAB), jnp.float32),
            pltpu.SemaphoreType.DMA,
            pltpu.SemaphoreType.DMA,
        ],
    )
    def k(xt_hbm, out_hbm, idb, buf0, buf1, sem0, sem1):
        wid = lax.axis_index("s") * 2 + lax.axis_index("c")
        d0_base = wid * SLAB

        pltpu.sync_copy(xt_hbm.at[:, pl.ds(d0_base, SLAB)], idb)

        iota = lax.iota(jnp.int32, L)
        z = jnp.zeros((L,), jnp.float32)

        def zero(buf):
            def zbody(i, _):
                for s in range(SLAB // L):
                    buf[i, pl.ds(s * L, L)] = z
                return 0

            lax.fori_loop(0, KPC, zbody, 0)

        def scatter(buf, q, value):
            j = q // CPJ
            kbase = (q % CPJ) * KPC
            vals = jnp.full((L,), value, jnp.float32)
            for g in range(GRP):
                ids = idb[j, pl.ds(g * L, L)]
                r = ids - kbase
                match = (r >= 0) & (r < KPC)
                plsc.store_scatter(
                    buf, [jnp.where(match, r, 0), iota + g * L], vals, mask=match
                )

        NB = 2
        bufs = (buf0, buf1)
        sems = (sem0, sem1)

        def dst(q):
            j = q // CPJ
            kbase = (q % CPJ) * KPC
            return out_hbm.at[j, pl.ds(kbase, KPC), pl.ds(d0_base, SLAB)]

        def fire(q, buf, sem):
            pltpu.async_copy(buf, dst(q), sem)

        def drain(q, buf, sem):
            pltpu.make_async_copy(buf, dst(q), sem).wait()

        for b in range(NB):
            zero(bufs[b])
            scatter(bufs[b], jnp.int32(b), 1.0)
            fire(jnp.int32(b), bufs[b], sems[b])

        def body(g, _):
            for b in range(NB):
                q = g + b
                drain(q - NB, bufs[b], sems[b])
                scatter(bufs[b], q - NB, 0.0)
                scatter(bufs[b], q, 1.0)
                fire(q, bufs[b], sems[b])
            return 0

        lax.fori_loop(1, NCHUNK // NB, lambda g, s: body(g * NB, s), 0)

        for b in range(NB):
            drain(jnp.int32(NCHUNK - NB + b), bufs[b], sems[b])

    return k


_sc_one_hot = _make_sc_one_hot()


def kernel(x):
    xt = jnp.transpose(x.astype(jnp.int32))
    out_t = _sc_one_hot(xt)
    return jnp.transpose(out_t, (2, 0, 1))


# speedup vs baseline: 1.0491x; 1.0078x over previous
import functools

import jax
import jax.numpy as jnp
from jax import lax
from jax.experimental import pallas as pl
from jax.experimental.pallas import tpu as pltpu
from jax.experimental.pallas import tpu_sc as plsc

NUM_CLASSES = 1000
ROWS = 4096
COLS = 20
L = 16
NW = 32
SLAB = ROWS // NW
TRPC = 25
KPC = 8 * TRPC
CPJ = NUM_CLASSES // KPC
NCHUNK = COLS * CPJ
GRP = SLAB // L


def _make_sc_one_hot():
    mesh = plsc.VectorSubcoreMesh(core_axis_name="c", subcore_axis_name="s")

    @functools.partial(
        pl.kernel,
        mesh=mesh,
        compiler_params=pltpu.CompilerParams(needs_layout_passes=False),
        out_type=jax.ShapeDtypeStruct((COLS, NUM_CLASSES, ROWS), jnp.float32),
        scratch_types=[
            pltpu.VMEM((COLS, SLAB), jnp.int32),
            pltpu.VMEM((KPC, SLAB), jnp.float32),
            pltpu.VMEM((KPC, SLAB), jnp.float32),
            pltpu.SemaphoreType.DMA,
            pltpu.SemaphoreType.DMA,
        ],
    )
    def k(xt_hbm, out_hbm, idb, buf0, buf1, sem0, sem1):
        wid = lax.axis_index("s") * 2 + lax.axis_index("c")
        d0_base = wid * SLAB

        idb_copy = pltpu.async_copy(xt_hbm.at[:, pl.ds(d0_base, SLAB)], idb, sem0)

        iota = lax.iota(jnp.int32, L)
        z = jnp.zeros((L,), jnp.float32)

        def zero(buf):
            def zbody(i, _):
                for s in range(SLAB // L):
                    buf[i, pl.ds(s * L, L)] = z
                return 0

            lax.fori_loop(0, KPC, zbody, 0)

        def scatter(buf, q, value):
            j = q // CPJ
            kbase = (q % CPJ) * KPC
            vals = jnp.full((L,), value, jnp.float32)
            for g in range(GRP):
                ids = idb[j, pl.ds(g * L, L)]
                r = ids - kbase
                match = (r >= 0) & (r < KPC)
                plsc.store_scatter(
                    buf, [jnp.where(match, r, 0), iota + g * L], vals, mask=match
                )

        NB = 2
        bufs = (buf0, buf1)
        sems = (sem0, sem1)

        def dst(q):
            j = q // CPJ
            kbase = (q % CPJ) * KPC
            return out_hbm.at[j, pl.ds(kbase, KPC), pl.ds(d0_base, SLAB)]

        def fire(q, buf, sem):
            pltpu.async_copy(buf, dst(q), sem)

        def drain(q, buf, sem):
            pltpu.make_async_copy(buf, dst(q), sem).wait()

        for b in range(NB):
            zero(bufs[b])
            if b == 0:
                idb_copy.wait()
            scatter(bufs[b], jnp.int32(b), 1.0)
            fire(jnp.int32(b), bufs[b], sems[b])

        def body(g, _):
            for b in range(NB):
                q = g + b
                drain(q - NB, bufs[b], sems[b])
                scatter(bufs[b], q - NB, 0.0)
                scatter(bufs[b], q, 1.0)
                fire(q, bufs[b], sems[b])
            return 0

        lax.fori_loop(1, NCHUNK // NB, lambda g, s: body(g * NB, s), 0)

        for b in range(NB):
            drain(jnp.int32(NCHUNK - NB + b), bufs[b], sems[b])

    return k


_sc_one_hot = _make_sc_one_hot()


def kernel(x):
    xt = jnp.transpose(x.astype(jnp.int32))
    out_t = _sc_one_hot(xt)
    return jnp.transpose(out_t, (2, 0, 1))
